# baseline (speedup 1.0000x reference)
import functools

import jax
import jax.numpy as jnp
import numpy as np
from jax import lax
from jax.experimental import pallas as pl
from jax.experimental.pallas import tpu as pltpu
from jax.experimental.pallas import tpu_sc as plsc

_FIELD_DIMS = [38462] * 26
_OFFSETS = np.concatenate([[0], np.cumsum(_FIELD_DIMS)[:-1]]).astype(np.int32)
_ROWS = int(np.sum(_FIELD_DIMS))
_D = 32
_B = 4096
_F = 26
_BF = _B * _F

_NC, _NS, _L = 2, 16, 16
_NW = _NC * _NS
_PER_W = _BF // _NW
_CHUNK = 128
_NCHUNK = _PER_W // _CHUNK


def _sc_body(xo_hbm, v_hbm, s_hbm, out_hbm, idx_v, rows_v, s_v, sem):
    wid = lax.axis_index("s") * _NC + lax.axis_index("c")
    base = wid * _PER_W

    pltpu.sync_copy(xo_hbm.at[wid], idx_v)
    pltpu.sync_copy(s_hbm, s_v)

    s0 = s_v[pl.ds(0, _L)]
    s1 = s_v[pl.ds(_L, _L)]
    t0 = 1.0 / (1.0 + jnp.exp(-s0))
    t1 = 1.0 / (1.0 + jnp.exp(-s1))
    nt0 = -t0
    nt1 = -t1

    def chunk_body(g, carry):
        pltpu.async_copy(v_hbm.at[idx_v.at[g]], rows_v, sem).wait()

        def row_body(i, c):
            r0 = rows_v[i, pl.ds(0, _L)]
            r1 = rows_v[i, pl.ds(_L, _L)]
            y0 = r0 - jnp.minimum(jnp.maximum(r0, nt0), t0)
            y1 = r1 - jnp.minimum(jnp.maximum(r1, nt1), t1)
            rows_v[i, pl.ds(0, _L)] = y0
            rows_v[i, pl.ds(_L, _L)] = y1
            return c

        lax.fori_loop(0, _CHUNK, row_body, 0, unroll=4)
        pltpu.sync_copy(rows_v, out_hbm.at[pl.ds(base + g * _CHUNK, _CHUNK)])
        return carry

    lax.fori_loop(0, _NCHUNK, chunk_body, 0)


@functools.partial(
    pl.kernel,
    out_type=jax.ShapeDtypeStruct((_BF, _D), jnp.float32),
    mesh=plsc.VectorSubcoreMesh(core_axis_name="c", subcore_axis_name="s"),
    compiler_params=pltpu.CompilerParams(use_tc_tiling_on_sc=False),
    scratch_types=[
        pltpu.VMEM((_NCHUNK, _CHUNK), jnp.int32),
        pltpu.VMEM((_CHUNK, _D), jnp.float32),
        pltpu.VMEM((_D,), jnp.float32),
        pltpu.SemaphoreType.DMA,
    ],
)
def _pep_embed_sc(xo_hbm, v_hbm, s_hbm, out_hbm, idx_v, rows_v, s_v, sem):
    _sc_body(xo_hbm, v_hbm, s_hbm, out_hbm, idx_v, rows_v, s_v, sem)


def kernel(x, v, s):
    xo = (x + jnp.asarray(_OFFSETS)[None, :]).reshape(_NW, _NCHUNK, _CHUNK)
    out = _pep_embed_sc(xo, v, s)
    return out.reshape(_B, _F, _D)
